# Initial kernel scaffold; baseline (speedup 1.0000x reference)
import functools

import jax
import jax.numpy as jnp
from jax import lax
from jax.experimental import pallas as pl
from jax.experimental.pallas import tpu as pltpu
from jax.experimental.pallas import tpu_sc as plsc

N = 10000
E = 320000
D = 128
G = 64
EPS = 1e-5
BNS = 1.0 / (1.0 + EPS) ** 0.5

NC = 2
NS = 16
NW = NC * NS
EPW = E // NW
CH = 80
NCH = EPW // CH
NPT = N // NS
ND2 = 10240
DPT = ND2 // NS

_MESH = plsc.VectorSubcoreMesh(core_axis_name="c", subcore_axis_name="s",
                               num_cores=NC, num_subcores=NS)


@functools.partial(
    pl.kernel,
    out_type=jax.ShapeDtypeStruct((NC, ND2), jnp.float32),
    mesh=_MESH,
    scratch_types=[
        pltpu.VMEM((NCH, CH), jnp.int32),
        pltpu.VMEM((CH,), jnp.float32),
        pltpu.VMEM((DPT,), jnp.float32),
    ],
)
def _deg_sc(dst3, ones_h, out, dstc, onesv, zv):
    c = lax.axis_index("c")
    s = lax.axis_index("s")
    w = s * NC + c

    def run(dsh):
        def zb(i, _):
            zv[pl.ds(i * 16, 16)] = jnp.zeros((16,), jnp.float32)
            return 0
        lax.fori_loop(0, DPT // 16, zb, 0)
        pltpu.sync_copy(zv, dsh.at[pl.ds(s * DPT, DPT)])
        pltpu.sync_copy(ones_h, onesv)
        pltpu.sync_copy(dst3.at[w], dstc)
        plsc.subcore_barrier()

        def body(i, _):
            pltpu.sync_copy(onesv, dsh.at[dstc.at[i]], add=True)
            return 0
        lax.fori_loop(0, NCH, body, 0)
        plsc.subcore_barrier()
        pltpu.sync_copy(dsh.at[pl.ds(s * DPT, DPT)],
                        out.at[c, pl.ds(s * DPT, DPT)])

    pl.run_scoped(run, pltpu.VMEM_SHARED((ND2,), jnp.float32))


@functools.partial(
    pl.kernel,
    out_type=jax.ShapeDtypeStruct((NC, N, D), jnp.float32),
    mesh=_MESH,
    scratch_types=[
        pltpu.VMEM((NCH, CH), jnp.int32),
        pltpu.VMEM((NCH, CH), jnp.int32),
        pltpu.VMEM((CH, D), jnp.float32),
        pltpu.SemaphoreType.DMA,
    ],
)
def _spmm_sc(u, src3, dst3, out, srcc, dstc, rows, sem):
    c = lax.axis_index("c")
    s = lax.axis_index("s")
    w = s * NC + c

    def run(acc):
        pltpu.sync_copy(u.at[pl.ds(s * NPT, NPT)], acc.at[pl.ds(s * NPT, NPT)])
        pltpu.sync_copy(src3.at[w], srcc)
        pltpu.sync_copy(dst3.at[w], dstc)
        plsc.subcore_barrier()

        def body(i, _):
            pltpu.async_copy(u.at[srcc.at[i]], rows, sem).wait()
            pltpu.sync_copy(rows, acc.at[dstc.at[i]], add=True)
            return 0
        lax.fori_loop(0, NCH, body, 0)
        plsc.subcore_barrier()
        pltpu.sync_copy(acc.at[pl.ds(s * NPT, NPT)],
                        out.at[c, pl.ds(s * NPT, NPT)])

    pl.run_scoped(run, pltpu.VMEM_SHARED((N, D), jnp.float32))


BT = 1000
NB = N // BT

_full = pl.BlockSpec((1, D), lambda i: (0, 0))
_rows = pl.BlockSpec((BT, D), lambda i: (i, 0))
_wmat = pl.BlockSpec((D, D), lambda i: (0, 0))
_col = pl.BlockSpec((BT, 1), lambda i: (i, 0))
_pblk = pl.BlockSpec((NC, BT, D), lambda i: (0, i, 0))


def _mm_body(x_ref, w_ref, o_ref):
    o_ref[...] = jnp.dot(x_ref[...], w_ref[...],
                         preferred_element_type=jnp.float32)


def _matmul(x, w):
    return pl.pallas_call(
        _mm_body,
        grid=(NB,),
        in_specs=[_rows, _wmat],
        out_specs=_rows,
        out_shape=jax.ShapeDtypeStruct((N, D), jnp.float32),
    )(x, w)


def _dinv_scale_body(dp_ref, h_ref, dinv_ref, u_ref):
    dv = lax.rsqrt(1.0 + dp_ref[0] + dp_ref[1])
    dinv_ref[...] = dv
    u_ref[...] = h_ref[...] * dv


def _dinv_scale(degpart, h):
    return pl.pallas_call(
        _dinv_scale_body,
        grid=(NB,),
        in_specs=[pl.BlockSpec((NC, BT, 1), lambda i: (0, i, 0)), _rows],
        out_specs=(_col, _rows),
        out_shape=(jax.ShapeDtypeStruct((N, 1), jnp.float32),
                   jax.ShapeDtypeStruct((N, D), jnp.float32)),
    )(degpart, h)


def _post_body(p_ref, u_ref, dinv_ref, b_ref, g_ref, be_ref, w_ref, id_ref,
               h_ref, un_ref, *, relu, residual):
    dv = dinv_ref[...]
    z = dv * (p_ref[0] + p_ref[1] - u_ref[...]) + b_ref[...]
    z = z * BNS * g_ref[...] + be_ref[...]
    if relu:
        z = jnp.maximum(z, 0.0)
    if residual:
        z = z + id_ref[...]
    h_ref[...] = z
    un_ref[...] = jnp.dot(z, w_ref[...],
                          preferred_element_type=jnp.float32) * dv


def _layer_post(p, u, dinv, b, g, be, w_next, identity, relu, residual):
    body = functools.partial(_post_body, relu=relu, residual=residual)
    return pl.pallas_call(
        body,
        grid=(NB,),
        in_specs=[_pblk, _rows, _col, _full, _full, _full, _wmat, _rows],
        out_specs=(_rows, _rows),
        out_shape=(jax.ShapeDtypeStruct((N, D), jnp.float32),
                   jax.ShapeDtypeStruct((N, D), jnp.float32)),
    )(p, u, dinv, b, g, be, w_next, identity)


def _final_body(p_ref, u_ref, dinv_ref, b_ref, g_ref, be_ref, batch_ref,
                h_ref, gemb_ref, sums, counts):
    i = pl.program_id(0)
    dv = dinv_ref[...]
    z = dv * (p_ref[0] + p_ref[1] - u_ref[...]) + b_ref[...]
    h = z * BNS * g_ref[...] + be_ref[...]
    h_ref[...] = h
    bid = batch_ref[...]
    gids = lax.broadcasted_iota(jnp.int32, (BT, G), 1)
    m = (bid == gids).astype(jnp.float32)
    dn = (((0,), (0,)), ((), ()))
    ps = lax.dot_general(m, h, dn, preferred_element_type=jnp.float32)
    pc = lax.dot_general(m, jnp.ones((BT, D), jnp.float32), dn,
                         preferred_element_type=jnp.float32)

    @pl.when(i == 0)
    def _():
        sums[...] = ps
        counts[...] = pc

    @pl.when(i > 0)
    def _():
        sums[...] += ps
        counts[...] += pc

    @pl.when(i == NB - 1)
    def _():
        gemb_ref[...] = sums[...] / jnp.maximum(counts[...], 1.0)


def _final_layer(p, u, dinv, b, g, be, batch2d):
    return pl.pallas_call(
        _final_body,
        grid=(NB,),
        in_specs=[_pblk, _rows, _col, _full, _full, _full,
                  pl.BlockSpec((BT, 1), lambda i: (i, 0))],
        out_specs=(_rows, pl.BlockSpec((G, D), lambda i: (0, 0))),
        out_shape=(jax.ShapeDtypeStruct((N, D), jnp.float32),
                   jax.ShapeDtypeStruct((G, D), jnp.float32)),
        scratch_shapes=[pltpu.VMEM((G, D), jnp.float32),
                        pltpu.VMEM((G, D), jnp.float32)],
    )(p, u, dinv, b, g, be, batch2d)


def kernel(x, edge_index, batch, W_in, b_in, g_in, be_in, W1, b1, g1, be1,
           W2, b2, g2, be2, W_out, b_out, g_out, be_out):
    src = edge_index[0].astype(jnp.int32).reshape(NW, NCH, CH)
    dst = edge_index[1].astype(jnp.int32).reshape(NW, NCH, CH)
    batch2d = batch.astype(jnp.int32).reshape(N, 1)
    ones_ch = jnp.ones((CH,), jnp.float32)

    degpart = _deg_sc(dst, ones_ch)
    degpart = degpart[:, :N].reshape(NC, N, 1)

    h0 = _matmul(x, W_in)
    dinv, u = _dinv_scale(degpart, h0)

    b_in2, g_in2, be_in2 = (a.reshape(1, D) for a in (b_in, g_in, be_in))
    b12, g12, be12 = (a.reshape(1, D) for a in (b1, g1, be1))
    b22, g22, be22 = (a.reshape(1, D) for a in (b2, g2, be2))
    b_o2, g_o2, be_o2 = (a.reshape(1, D) for a in (b_out, g_out, be_out))

    zero_id = u

    p = _spmm_sc(u, src, dst)
    h1, u = _layer_post(p, u, dinv, b_in2, g_in2, be_in2, W1, zero_id,
                        relu=True, residual=False)
    p = _spmm_sc(u, src, dst)
    h2, u = _layer_post(p, u, dinv, b12, g12, be12, W2, h1,
                        relu=True, residual=True)
    p = _spmm_sc(u, src, dst)
    h3, u = _layer_post(p, u, dinv, b22, g22, be22, W_out, h2,
                        relu=True, residual=True)
    p = _spmm_sc(u, src, dst)
    node_embeddings, graph_embedding = _final_layer(
        p, u, dinv, b_o2, g_o2, be_o2, batch2d)
    return (node_embeddings, graph_embedding)


# speedup vs baseline: 15.0253x; 15.0253x over previous
import functools

import jax
import jax.numpy as jnp
from jax import lax
from jax.experimental import pallas as pl
from jax.experimental.pallas import tpu as pltpu
from jax.experimental.pallas import tpu_sc as plsc

N = 10000
E = 320000
D = 128
G = 64
EPS = 1e-5
BNS = 1.0 / (1.0 + EPS) ** 0.5

NC = 2
NS = 16
NW = NC * NS
EPW = E // NW
CH = 80
NCH = EPW // CH
N2 = 10240
NPT = N2 // NS


@functools.lru_cache(maxsize=1)
def _sc_kernels():
    mesh = plsc.VectorSubcoreMesh(core_axis_name="c", subcore_axis_name="s",
                                  num_cores=NC, num_subcores=NS)

    @functools.partial(
        pl.kernel,
        out_type=jax.ShapeDtypeStruct((NC, N2), jnp.float32),
        mesh=mesh,
        scratch_types=[
            pltpu.VMEM_SHARED((N2,), jnp.float32),
            pltpu.VMEM((NCH, CH), jnp.int32),
            pltpu.VMEM((CH,), jnp.float32),
            pltpu.VMEM((NPT,), jnp.float32),
        ],
    )
    def deg_sc(dst3, ones_h, out, dsh, dstc, onesv, zv):
        c = lax.axis_index("c")
        s = lax.axis_index("s")
        w = s * NC + c

        def zb(i, _):
            zv[pl.ds(i * 16, 16)] = jnp.zeros((16,), jnp.float32)
            return 0
        lax.fori_loop(0, NPT // 16, zb, 0)
        pltpu.sync_copy(zv, dsh.at[pl.ds(s * NPT, NPT)])
        pltpu.sync_copy(ones_h, onesv)
        pltpu.sync_copy(dst3.at[w], dstc)
        plsc.subcore_barrier()

        def body(i, _):
            pltpu.sync_copy(onesv, dsh.at[dstc.at[i]], add=True)
            return 0
        lax.fori_loop(0, NCH, body, 0)
        plsc.subcore_barrier()
        pltpu.sync_copy(dsh.at[pl.ds(s * NPT, NPT)],
                        out.at[c, pl.ds(s * NPT, NPT)])

    @functools.partial(
        pl.kernel,
        out_type=jax.ShapeDtypeStruct((NC, N2, D), jnp.float32),
        mesh=mesh,
        scratch_types=[
            pltpu.VMEM_SHARED((N2, D), jnp.float32),
            pltpu.VMEM((NCH, CH), jnp.int32),
            pltpu.VMEM((NCH, CH), jnp.int32),
            pltpu.VMEM((CH, D), jnp.float32),
            pltpu.SemaphoreType.DMA,
        ],
    )
    def spmm_sc(u, src3, dst3, out, acc, srcc, dstc, rows, sem):
        c = lax.axis_index("c")
        s = lax.axis_index("s")
        w = s * NC + c

        pltpu.sync_copy(u.at[pl.ds(s * NPT, NPT)], acc.at[pl.ds(s * NPT, NPT)])
        pltpu.sync_copy(src3.at[w], srcc)
        pltpu.sync_copy(dst3.at[w], dstc)
        plsc.subcore_barrier()

        def body(i, _):
            pltpu.async_copy(u.at[srcc.at[i]], rows, sem).wait()
            pltpu.sync_copy(rows, acc.at[dstc.at[i]], add=True)
            return 0
        lax.fori_loop(0, NCH, body, 0)
        plsc.subcore_barrier()
        pltpu.sync_copy(acc.at[pl.ds(s * NPT, NPT)],
                        out.at[c, pl.ds(s * NPT, NPT)])

    return deg_sc, spmm_sc


def _deg_sc(dst3, ones_h):
    return _sc_kernels()[0](dst3, ones_h)


def _spmm_sc(u, src3, dst3):
    return _sc_kernels()[1](u, src3, dst3)


def _spmm_jnp(u, src3, dst3):
    s_idx = src3.reshape(-1)
    d_idx = dst3.reshape(-1)
    agg = jnp.zeros((N2, D), jnp.float32).at[d_idx].add(u[s_idx])
    return jnp.stack([u + agg, u + jnp.zeros_like(agg)])


def _deg_jnp(dst3, ones_ch):
    d_idx = dst3.reshape(-1)
    deg = jnp.zeros((N2,), jnp.float32).at[d_idx].add(1.0)
    return jnp.stack([deg, jnp.zeros_like(deg)])


BT = 1024
NB = N2 // BT

_full = pl.BlockSpec((1, D), lambda i: (0, 0))
_rows = pl.BlockSpec((BT, D), lambda i: (i, 0))
_wmat = pl.BlockSpec((D, D), lambda i: (0, 0))
_col = pl.BlockSpec((BT, 1), lambda i: (i, 0))
_pblk = pl.BlockSpec((NC, BT, D), lambda i: (0, i, 0))


def _mm_body(x_ref, w_ref, o_ref):
    o_ref[...] = jnp.dot(x_ref[...], w_ref[...],
                         preferred_element_type=jnp.float32)


def _matmul(x, w):
    return pl.pallas_call(
        _mm_body,
        grid=(NB,),
        in_specs=[_rows, _wmat],
        out_specs=_rows,
        out_shape=jax.ShapeDtypeStruct((N2, D), jnp.float32),
    )(x, w)


def _dinv_scale_body(dp_ref, h_ref, dinv_ref, u_ref):
    dv = lax.rsqrt(1.0 + dp_ref[0] + dp_ref[1])
    dinv_ref[...] = dv
    u_ref[...] = h_ref[...] * dv


def _dinv_scale(degpart, h):
    return pl.pallas_call(
        _dinv_scale_body,
        grid=(NB,),
        in_specs=[pl.BlockSpec((NC, BT, 1), lambda i: (0, i, 0)), _rows],
        out_specs=(_col, _rows),
        out_shape=(jax.ShapeDtypeStruct((N2, 1), jnp.float32),
                   jax.ShapeDtypeStruct((N2, D), jnp.float32)),
    )(degpart, h)


def _post_body(p_ref, u_ref, dinv_ref, b_ref, g_ref, be_ref, w_ref, id_ref,
               h_ref, un_ref, *, relu, residual):
    dv = dinv_ref[...]
    z = dv * (p_ref[0] + p_ref[1] - u_ref[...]) + b_ref[...]
    z = z * BNS * g_ref[...] + be_ref[...]
    if relu:
        z = jnp.maximum(z, 0.0)
    if residual:
        z = z + id_ref[...]
    h_ref[...] = z
    un_ref[...] = jnp.dot(z, w_ref[...],
                          preferred_element_type=jnp.float32) * dv


def _layer_post(p, u, dinv, b, g, be, w_next, identity, relu, residual):
    body = functools.partial(_post_body, relu=relu, residual=residual)
    return pl.pallas_call(
        body,
        grid=(NB,),
        in_specs=[_pblk, _rows, _col, _full, _full, _full, _wmat, _rows],
        out_specs=(_rows, _rows),
        out_shape=(jax.ShapeDtypeStruct((N2, D), jnp.float32),
                   jax.ShapeDtypeStruct((N2, D), jnp.float32)),
    )(p, u, dinv, b, g, be, w_next, identity)


def _final_body(p_ref, u_ref, dinv_ref, b_ref, g_ref, be_ref, batch_ref,
                h_ref, gemb_ref, sums, counts):
    i = pl.program_id(0)
    dv = dinv_ref[...]
    z = dv * (p_ref[0] + p_ref[1] - u_ref[...]) + b_ref[...]
    h = z * BNS * g_ref[...] + be_ref[...]
    h_ref[...] = h
    bid = batch_ref[...]
    gids = lax.broadcasted_iota(jnp.int32, (BT, G), 1)
    m = (bid == gids).astype(jnp.float32)
    dn = (((0,), (0,)), ((), ()))
    ps = lax.dot_general(m, h, dn, preferred_element_type=jnp.float32)
    pc = lax.dot_general(m, jnp.ones((BT, D), jnp.float32), dn,
                         preferred_element_type=jnp.float32)

    @pl.when(i == 0)
    def _():
        sums[...] = ps
        counts[...] = pc

    @pl.when(i > 0)
    def _():
        sums[...] += ps
        counts[...] += pc

    @pl.when(i == NB - 1)
    def _():
        gemb_ref[...] = sums[...] / jnp.maximum(counts[...], 1.0)


def _final_layer(p, u, dinv, b, g, be, batch2d):
    return pl.pallas_call(
        _final_body,
        grid=(NB,),
        in_specs=[_pblk, _rows, _col, _full, _full, _full,
                  pl.BlockSpec((BT, 1), lambda i: (i, 0))],
        out_specs=(_rows, pl.BlockSpec((G, D), lambda i: (0, 0))),
        out_shape=(jax.ShapeDtypeStruct((N2, D), jnp.float32),
                   jax.ShapeDtypeStruct((G, D), jnp.float32)),
        scratch_shapes=[pltpu.VMEM((G, D), jnp.float32),
                        pltpu.VMEM((G, D), jnp.float32)],
    )(p, u, dinv, b, g, be, batch2d)


def kernel(x, edge_index, batch, W_in, b_in, g_in, be_in, W1, b1, g1, be1,
           W2, b2, g2, be2, W_out, b_out, g_out, be_out):
    src = edge_index[0].astype(jnp.int32).reshape(NW, NCH, CH)
    dst = edge_index[1].astype(jnp.int32).reshape(NW, NCH, CH)
    batch2d = jnp.pad(batch.astype(jnp.int32), (0, N2 - N),
                      constant_values=G).reshape(N2, 1)
    xp = jnp.pad(x, ((0, N2 - N), (0, 0)))
    ones_ch = jnp.ones((CH,), jnp.float32)

    degpart = _deg_sc(dst, ones_ch).reshape(NC, N2, 1)

    h0 = _matmul(xp, W_in)
    dinv, u = _dinv_scale(degpart, h0)

    b_in2, g_in2, be_in2 = (a.reshape(1, D) for a in (b_in, g_in, be_in))
    b12, g12, be12 = (a.reshape(1, D) for a in (b1, g1, be1))
    b22, g22, be22 = (a.reshape(1, D) for a in (b2, g2, be2))
    b_o2, g_o2, be_o2 = (a.reshape(1, D) for a in (b_out, g_out, be_out))

    zero_id = u

    p = _spmm_sc(u, src, dst)
    h1, u = _layer_post(p, u, dinv, b_in2, g_in2, be_in2, W1, zero_id,
                        relu=True, residual=False)
    p = _spmm_sc(u, src, dst)
    h2, u = _layer_post(p, u, dinv, b12, g12, be12, W2, h1,
                        relu=True, residual=True)
    p = _spmm_sc(u, src, dst)
    h3, u = _layer_post(p, u, dinv, b22, g22, be22, W_out, h2,
                        relu=True, residual=True)
    p = _spmm_sc(u, src, dst)
    h4, graph_embedding = _final_layer(
        p, u, dinv, b_o2, g_o2, be_o2, batch2d)
    return (h4[:N], graph_embedding)


# speedup vs baseline: 23.7077x; 1.5779x over previous
import functools

import jax
import jax.numpy as jnp
from jax import lax
from jax.experimental import pallas as pl
from jax.experimental.pallas import tpu as pltpu
from jax.experimental.pallas import tpu_sc as plsc

N = 10000
E = 320000
D = 128
G = 64
EPS = 1e-5
BNS = 1.0 / (1.0 + EPS) ** 0.5

NC = 2
NS = 16
NW = NC * NS
EPW = E // NW
CH = 80
NCH = EPW // CH
N2 = 10240
NPT = N2 // NS


@functools.lru_cache(maxsize=1)
def _sc_kernels():
    mesh = plsc.VectorSubcoreMesh(core_axis_name="c", subcore_axis_name="s",
                                  num_cores=NC, num_subcores=NS)

    @functools.partial(
        pl.kernel,
        out_type=jax.ShapeDtypeStruct((NC, N2), jnp.float32),
        mesh=mesh,
        scratch_types=[
            pltpu.VMEM_SHARED((N2,), jnp.float32),
            pltpu.VMEM((NCH, CH), jnp.int32),
            pltpu.VMEM((CH,), jnp.float32),
            pltpu.VMEM((NPT,), jnp.float32),
        ],
    )
    def deg_sc(dst3, ones_h, out, dsh, dstc, onesv, zv):
        c = lax.axis_index("c")
        s = lax.axis_index("s")
        w = s * NC + c

        def zb(i, _):
            zv[pl.ds(i * 16, 16)] = jnp.zeros((16,), jnp.float32)
            return 0
        lax.fori_loop(0, NPT // 16, zb, 0)
        pltpu.sync_copy(zv, dsh.at[pl.ds(s * NPT, NPT)])
        pltpu.sync_copy(ones_h, onesv)
        pltpu.sync_copy(dst3.at[w], dstc)
        plsc.subcore_barrier()

        def body(i, _):
            pltpu.sync_copy(onesv, dsh.at[dstc.at[i]], add=True)
            return 0
        lax.fori_loop(0, NCH, body, 0)
        plsc.subcore_barrier()
        pltpu.sync_copy(dsh.at[pl.ds(s * NPT, NPT)],
                        out.at[c, pl.ds(s * NPT, NPT)])

    @functools.partial(
        pl.kernel,
        out_type=jax.ShapeDtypeStruct((NC, N2, D), jnp.float32),
        mesh=mesh,
        scratch_types=[
            pltpu.VMEM_SHARED((N2, D), jnp.float32),
            pltpu.VMEM((EPW,), jnp.int32),
            pltpu.VMEM((NCH, CH), jnp.int32),
            pltpu.VMEM((CH, D), jnp.float32),
            pltpu.VMEM((CH, D), jnp.float32),
            pltpu.SemaphoreType.DMA,
            pltpu.SemaphoreType.DMA,
        ],
    )
    def spmm_sc(u, src2, dst3, out, acc, srcc, dstc, rows0, rows1,
                sem0, sem1):
        c = lax.axis_index("c")
        s = lax.axis_index("s")
        w = s * NC + c

        pltpu.sync_copy(u.at[pl.ds(s * NPT, NPT)], acc.at[pl.ds(s * NPT, NPT)])
        pltpu.sync_copy(src2.at[w], srcc)
        pltpu.sync_copy(dst3.at[w], dstc)
        plsc.subcore_barrier()

        def sidx(i):
            return srcc.at[pl.ds(i * CH, CH)]

        pltpu.async_copy(u.at[sidx(0)], rows0, sem0)

        def body(j, _):
            i0 = 2 * j
            g1 = pltpu.async_copy(u.at[sidx(i0 + 1)], rows1, sem1)
            pltpu.make_async_copy(u.at[sidx(i0)], rows0, sem0).wait()
            pltpu.sync_copy(rows0, acc.at[dstc.at[i0]], add=True)
            pltpu.async_copy(u.at[sidx(i0 + 2)], rows0, sem0)
            g1.wait()
            pltpu.sync_copy(rows1, acc.at[dstc.at[i0 + 1]], add=True)
            return 0
        lax.fori_loop(0, (NCH - 1) // 2, body, 0)
        pltpu.make_async_copy(u.at[sidx(NCH - 1)], rows0, sem0).wait()
        pltpu.sync_copy(rows0, acc.at[dstc.at[NCH - 1]], add=True)

        plsc.subcore_barrier()
        pltpu.sync_copy(acc.at[pl.ds(s * NPT, NPT)],
                        out.at[c, pl.ds(s * NPT, NPT)])

    return deg_sc, spmm_sc


def _deg_sc(dst3, ones_h):
    return _sc_kernels()[0](dst3, ones_h)


def _spmm_sc(u, src3, dst3):
    return _sc_kernels()[1](u, src3, dst3)


def _spmm_jnp(u, src3, dst3):
    s_idx = src3.reshape(-1)
    d_idx = dst3.reshape(-1)
    agg = jnp.zeros((N2, D), jnp.float32).at[d_idx].add(u[s_idx])
    return jnp.stack([u + agg, u + jnp.zeros_like(agg)])


def _deg_jnp(dst3, ones_ch):
    d_idx = dst3.reshape(-1)
    deg = jnp.zeros((N2,), jnp.float32).at[d_idx].add(1.0)
    return jnp.stack([deg, jnp.zeros_like(deg)])


BT = 1024
NB = N2 // BT

_full = pl.BlockSpec((1, D), lambda i: (0, 0))
_rows = pl.BlockSpec((BT, D), lambda i: (i, 0))
_wmat = pl.BlockSpec((D, D), lambda i: (0, 0))
_col = pl.BlockSpec((BT, 1), lambda i: (i, 0))
_pblk = pl.BlockSpec((NC, BT, D), lambda i: (0, i, 0))


def _mm_body(x_ref, w_ref, o_ref):
    o_ref[...] = jnp.dot(x_ref[...], w_ref[...],
                         preferred_element_type=jnp.float32)


def _matmul(x, w):
    return pl.pallas_call(
        _mm_body,
        grid=(NB,),
        in_specs=[_rows, _wmat],
        out_specs=_rows,
        out_shape=jax.ShapeDtypeStruct((N2, D), jnp.float32),
    )(x, w)


def _dinv_scale_body(dp_ref, h_ref, dinv_ref, u_ref):
    dv = lax.rsqrt(1.0 + dp_ref[0] + dp_ref[1])
    dinv_ref[...] = dv
    u_ref[...] = h_ref[...] * dv


def _dinv_scale(degpart, h):
    return pl.pallas_call(
        _dinv_scale_body,
        grid=(NB,),
        in_specs=[pl.BlockSpec((NC, BT, 1), lambda i: (0, i, 0)), _rows],
        out_specs=(_col, _rows),
        out_shape=(jax.ShapeDtypeStruct((N2, 1), jnp.float32),
                   jax.ShapeDtypeStruct((N2, D), jnp.float32)),
    )(degpart, h)


def _post_body(p_ref, u_ref, dinv_ref, b_ref, g_ref, be_ref, w_ref, id_ref,
               h_ref, un_ref, *, relu, residual):
    dv = dinv_ref[...]
    z = dv * (p_ref[0] + p_ref[1] - u_ref[...]) + b_ref[...]
    z = z * BNS * g_ref[...] + be_ref[...]
    if relu:
        z = jnp.maximum(z, 0.0)
    if residual:
        z = z + id_ref[...]
    h_ref[...] = z
    un_ref[...] = jnp.dot(z, w_ref[...],
                          preferred_element_type=jnp.float32) * dv


def _layer_post(p, u, dinv, b, g, be, w_next, identity, relu, residual):
    body = functools.partial(_post_body, relu=relu, residual=residual)
    return pl.pallas_call(
        body,
        grid=(NB,),
        in_specs=[_pblk, _rows, _col, _full, _full, _full, _wmat, _rows],
        out_specs=(_rows, _rows),
        out_shape=(jax.ShapeDtypeStruct((N2, D), jnp.float32),
                   jax.ShapeDtypeStruct((N2, D), jnp.float32)),
    )(p, u, dinv, b, g, be, w_next, identity)


def _final_body(p_ref, u_ref, dinv_ref, b_ref, g_ref, be_ref, batch_ref,
                h_ref, gemb_ref, sums, counts):
    i = pl.program_id(0)
    dv = dinv_ref[...]
    z = dv * (p_ref[0] + p_ref[1] - u_ref[...]) + b_ref[...]
    h = z * BNS * g_ref[...] + be_ref[...]
    h_ref[...] = h
    bid = batch_ref[...]
    gids = lax.broadcasted_iota(jnp.int32, (BT, G), 1)
    m = (bid == gids).astype(jnp.float32)
    dn = (((0,), (0,)), ((), ()))
    ps = lax.dot_general(m, h, dn, preferred_element_type=jnp.float32)
    pc = lax.dot_general(m, jnp.ones((BT, D), jnp.float32), dn,
                         preferred_element_type=jnp.float32)

    @pl.when(i == 0)
    def _():
        sums[...] = ps
        counts[...] = pc

    @pl.when(i > 0)
    def _():
        sums[...] += ps
        counts[...] += pc

    @pl.when(i == NB - 1)
    def _():
        gemb_ref[...] = sums[...] / jnp.maximum(counts[...], 1.0)


def _final_layer(p, u, dinv, b, g, be, batch2d):
    return pl.pallas_call(
        _final_body,
        grid=(NB,),
        in_specs=[_pblk, _rows, _col, _full, _full, _full,
                  pl.BlockSpec((BT, 1), lambda i: (i, 0))],
        out_specs=(_rows, pl.BlockSpec((G, D), lambda i: (0, 0))),
        out_shape=(jax.ShapeDtypeStruct((N2, D), jnp.float32),
                   jax.ShapeDtypeStruct((G, D), jnp.float32)),
        scratch_shapes=[pltpu.VMEM((G, D), jnp.float32),
                        pltpu.VMEM((G, D), jnp.float32)],
    )(p, u, dinv, b, g, be, batch2d)


def kernel(x, edge_index, batch, W_in, b_in, g_in, be_in, W1, b1, g1, be1,
           W2, b2, g2, be2, W_out, b_out, g_out, be_out):
    src = edge_index[0].astype(jnp.int32).reshape(NW, EPW)
    dst = edge_index[1].astype(jnp.int32).reshape(NW, NCH, CH)
    batch2d = jnp.pad(batch.astype(jnp.int32), (0, N2 - N),
                      constant_values=G).reshape(N2, 1)
    xp = jnp.pad(x, ((0, N2 - N), (0, 0)))
    ones_ch = jnp.ones((CH,), jnp.float32)

    degpart = _deg_sc(dst, ones_ch).reshape(NC, N2, 1)

    h0 = _matmul(xp, W_in)
    dinv, u = _dinv_scale(degpart, h0)

    b_in2, g_in2, be_in2 = (a.reshape(1, D) for a in (b_in, g_in, be_in))
    b12, g12, be12 = (a.reshape(1, D) for a in (b1, g1, be1))
    b22, g22, be22 = (a.reshape(1, D) for a in (b2, g2, be2))
    b_o2, g_o2, be_o2 = (a.reshape(1, D) for a in (b_out, g_out, be_out))

    zero_id = u

    p = _spmm_sc(u, src, dst)
    h1, u = _layer_post(p, u, dinv, b_in2, g_in2, be_in2, W1, zero_id,
                        relu=True, residual=False)
    p = _spmm_sc(u, src, dst)
    h2, u = _layer_post(p, u, dinv, b12, g12, be12, W2, h1,
                        relu=True, residual=True)
    p = _spmm_sc(u, src, dst)
    h3, u = _layer_post(p, u, dinv, b22, g22, be22, W_out, h2,
                        relu=True, residual=True)
    p = _spmm_sc(u, src, dst)
    h4, graph_embedding = _final_layer(
        p, u, dinv, b_o2, g_o2, be_o2, batch2d)
    return (h4[:N], graph_embedding)


# speedup vs baseline: 23.7490x; 1.0017x over previous
import functools

import jax
import jax.numpy as jnp
from jax import lax
from jax.experimental import pallas as pl
from jax.experimental.pallas import tpu as pltpu
from jax.experimental.pallas import tpu_sc as plsc

N = 10000
E = 320000
D = 128
G = 64
EPS = 1e-5
BNS = 1.0 / (1.0 + EPS) ** 0.5

NC = 2
NS = 16
NW = NC * NS
EPW = E // NW
CH = 80
NCH = EPW // CH
N2 = 10240
NPT = N2 // NS


@functools.lru_cache(maxsize=1)
def _sc_kernels():
    mesh = plsc.VectorSubcoreMesh(core_axis_name="c", subcore_axis_name="s",
                                  num_cores=NC, num_subcores=NS)

    @functools.partial(
        pl.kernel,
        out_type=jax.ShapeDtypeStruct((NC, N2), jnp.float32),
        mesh=mesh,
        scratch_types=[
            pltpu.VMEM_SHARED((N2,), jnp.float32),
            pltpu.VMEM((NCH, CH), jnp.int32),
            pltpu.VMEM((CH,), jnp.float32),
            pltpu.VMEM((NPT,), jnp.float32),
        ],
    )
    def deg_sc(dst3, ones_h, out, dsh, dstc, onesv, zv):
        c = lax.axis_index("c")
        s = lax.axis_index("s")
        w = s * NC + c

        def zb(i, _):
            zv[pl.ds(i * 16, 16)] = jnp.zeros((16,), jnp.float32)
            return 0
        lax.fori_loop(0, NPT // 16, zb, 0)
        pltpu.sync_copy(zv, dsh.at[pl.ds(s * NPT, NPT)])
        pltpu.sync_copy(ones_h, onesv)
        pltpu.sync_copy(dst3.at[w], dstc)
        plsc.subcore_barrier()

        def body(i, _):
            pltpu.sync_copy(onesv, dsh.at[dstc.at[i]], add=True)
            return 0
        lax.fori_loop(0, NCH, body, 0)
        plsc.subcore_barrier()
        pltpu.sync_copy(dsh.at[pl.ds(s * NPT, NPT)],
                        out.at[c, pl.ds(s * NPT, NPT)])

    @functools.partial(
        pl.kernel,
        out_type=jax.ShapeDtypeStruct((NC, N2, D), jnp.float32),
        mesh=mesh,
        scratch_types=[
            pltpu.VMEM_SHARED((N2, D), jnp.float32),
            pltpu.VMEM((EPW,), jnp.int32),
            pltpu.VMEM((NCH, CH), jnp.int32),
            pltpu.VMEM((CH, D), jnp.float32),
            pltpu.VMEM((CH, D), jnp.float32),
            pltpu.SemaphoreType.DMA,
            pltpu.SemaphoreType.DMA,
        ],
    )
    def spmm_sc(u, src2, dst3, out, acc, srcc, dstc, rows0, rows1,
                sem0, sem1):
        c = lax.axis_index("c")
        s = lax.axis_index("s")
        w = s * NC + c

        pltpu.sync_copy(u.at[pl.ds(s * NPT, NPT)], acc.at[pl.ds(s * NPT, NPT)])
        pltpu.sync_copy(src2.at[w], srcc)
        pltpu.sync_copy(dst3.at[w], dstc)
        plsc.subcore_barrier()

        def sidx(i):
            return srcc.at[pl.ds(i * CH, CH)]

        pltpu.async_copy(u.at[sidx(0)], rows0, sem0)

        def body(j, _):
            i0 = 2 * j
            g1 = pltpu.async_copy(u.at[sidx(i0 + 1)], rows1, sem1)
            pltpu.make_async_copy(u.at[sidx(i0)], rows0, sem0).wait()
            pltpu.sync_copy(rows0, acc.at[dstc.at[i0]], add=True)
            pltpu.async_copy(u.at[sidx(i0 + 2)], rows0, sem0)
            g1.wait()
            pltpu.sync_copy(rows1, acc.at[dstc.at[i0 + 1]], add=True)
            return 0
        lax.fori_loop(0, (NCH - 1) // 2, body, 0)
        pltpu.make_async_copy(u.at[sidx(NCH - 1)], rows0, sem0).wait()
        pltpu.sync_copy(rows0, acc.at[dstc.at[NCH - 1]], add=True)

        plsc.subcore_barrier()
        pltpu.sync_copy(acc.at[pl.ds(s * NPT, NPT)],
                        out.at[c, pl.ds(s * NPT, NPT)])

    return deg_sc, spmm_sc


def _deg_sc(dst3, ones_h):
    return _sc_kernels()[0](dst3, ones_h)


def _spmm_sc(u, src3, dst3):
    return _sc_kernels()[1](u, src3, dst3)


BT = 1024
NB = N2 // BT

_full = pl.BlockSpec((1, D), lambda i: (0, 0))
_rows = pl.BlockSpec((BT, D), lambda i: (i, 0))
_wmat = pl.BlockSpec((D, D), lambda i: (0, 0))
_col = pl.BlockSpec((BT, 1), lambda i: (i, 0))
_pblk = pl.BlockSpec((NC, BT, D), lambda i: (0, i, 0))


def _mm_scale_body(x_ref, w_ref, dp_ref, dinv_ref, u_ref):
    dv = lax.rsqrt(1.0 + dp_ref[0] + dp_ref[1])
    dinv_ref[...] = dv
    u_ref[...] = jnp.dot(x_ref[...], w_ref[...],
                         preferred_element_type=jnp.float32) * dv


def _mm_scale(x, w, degpart):
    return pl.pallas_call(
        _mm_scale_body,
        grid=(NB,),
        in_specs=[_rows, _wmat, pl.BlockSpec((NC, BT, 1), lambda i: (0, i, 0))],
        out_specs=(_col, _rows),
        out_shape=(jax.ShapeDtypeStruct((N2, 1), jnp.float32),
                   jax.ShapeDtypeStruct((N2, D), jnp.float32)),
    )(x, w, degpart)


def _post_body(p_ref, u_ref, dinv_ref, b_ref, g_ref, be_ref, w_ref, id_ref,
               h_ref, un_ref, *, relu, residual):
    dv = dinv_ref[...]
    z = dv * (p_ref[0] + p_ref[1] - u_ref[...]) + b_ref[...]
    z = z * BNS * g_ref[...] + be_ref[...]
    if relu:
        z = jnp.maximum(z, 0.0)
    if residual:
        z = z + id_ref[...]
    h_ref[...] = z
    un_ref[...] = jnp.dot(z, w_ref[...],
                          preferred_element_type=jnp.float32) * dv


def _layer_post(p, u, dinv, b, g, be, w_next, identity, relu, residual):
    body = functools.partial(_post_body, relu=relu, residual=residual)
    return pl.pallas_call(
        body,
        grid=(NB,),
        in_specs=[_pblk, _rows, _col, _full, _full, _full, _wmat, _rows],
        out_specs=(_rows, _rows),
        out_shape=(jax.ShapeDtypeStruct((N2, D), jnp.float32),
                   jax.ShapeDtypeStruct((N2, D), jnp.float32)),
    )(p, u, dinv, b, g, be, w_next, identity)


def _final_body(p_ref, u_ref, dinv_ref, b_ref, g_ref, be_ref, batch_ref,
                h_ref, gemb_ref, sums, counts):
    i = pl.program_id(0)
    dv = dinv_ref[...]
    z = dv * (p_ref[0] + p_ref[1] - u_ref[...]) + b_ref[...]
    h = z * BNS * g_ref[...] + be_ref[...]
    h_ref[...] = h
    bid = batch_ref[...]
    gids = lax.broadcasted_iota(jnp.int32, (BT, G), 1)
    m = (bid == gids).astype(jnp.float32)
    dn = (((0,), (0,)), ((), ()))
    ps = lax.dot_general(m, h, dn, preferred_element_type=jnp.float32)
    pc = lax.dot_general(m, jnp.ones((BT, D), jnp.float32), dn,
                         preferred_element_type=jnp.float32)

    @pl.when(i == 0)
    def _():
        sums[...] = ps
        counts[...] = pc

    @pl.when(i > 0)
    def _():
        sums[...] += ps
        counts[...] += pc

    @pl.when(i == NB - 1)
    def _():
        gemb_ref[...] = sums[...] / jnp.maximum(counts[...], 1.0)


def _final_layer(p, u, dinv, b, g, be, batch2d):
    return pl.pallas_call(
        _final_body,
        grid=(NB,),
        in_specs=[_pblk, _rows, _col, _full, _full, _full,
                  pl.BlockSpec((BT, 1), lambda i: (i, 0))],
        out_specs=(_rows, pl.BlockSpec((G, D), lambda i: (0, 0))),
        out_shape=(jax.ShapeDtypeStruct((N2, D), jnp.float32),
                   jax.ShapeDtypeStruct((G, D), jnp.float32)),
        scratch_shapes=[pltpu.VMEM((G, D), jnp.float32),
                        pltpu.VMEM((G, D), jnp.float32)],
    )(p, u, dinv, b, g, be, batch2d)


def kernel(x, edge_index, batch, W_in, b_in, g_in, be_in, W1, b1, g1, be1,
           W2, b2, g2, be2, W_out, b_out, g_out, be_out):
    src = edge_index[0].astype(jnp.int32).reshape(NW, EPW)
    dst = edge_index[1].astype(jnp.int32).reshape(NW, NCH, CH)
    batch2d = jnp.pad(batch.astype(jnp.int32), (0, N2 - N),
                      constant_values=G).reshape(N2, 1)
    xp = jnp.pad(x, ((0, N2 - N), (0, 0)))
    ones_ch = jnp.ones((CH,), jnp.float32)

    degpart = _deg_sc(dst, ones_ch).reshape(NC, N2, 1)
    dinv, u = _mm_scale(xp, W_in, degpart)

    b_in2, g_in2, be_in2 = (a.reshape(1, D) for a in (b_in, g_in, be_in))
    b12, g12, be12 = (a.reshape(1, D) for a in (b1, g1, be1))
    b22, g22, be22 = (a.reshape(1, D) for a in (b2, g2, be2))
    b_o2, g_o2, be_o2 = (a.reshape(1, D) for a in (b_out, g_out, be_out))

    zero_id = u

    p = _spmm_sc(u, src, dst)
    h1, u = _layer_post(p, u, dinv, b_in2, g_in2, be_in2, W1, zero_id,
                        relu=True, residual=False)
    p = _spmm_sc(u, src, dst)
    h2, u = _layer_post(p, u, dinv, b12, g12, be12, W2, h1,
                        relu=True, residual=True)
    p = _spmm_sc(u, src, dst)
    h3, u = _layer_post(p, u, dinv, b22, g22, be22, W_out, h2,
                        relu=True, residual=True)
    p = _spmm_sc(u, src, dst)
    h4, graph_embedding = _final_layer(
        p, u, dinv, b_o2, g_o2, be_o2, batch2d)
    return (h4[:N], graph_embedding)
